# baseline (speedup 1.0000x reference)
import jax
import jax.numpy as jnp
from jax.experimental import pallas as pl

_BN = 4096


def _fused_kernel(x_ref, w_ref, b_ref, y_ref, ep_ref, gl_ref):
    acc = jnp.dot(x_ref[...], w_ref[...], preferred_element_type=jnp.float32)
    y_ref[...] = acc + b_ref[...]
    ep_ref[...] = jnp.zeros_like(ep_ref)
    gl_ref[...] = jnp.zeros_like(gl_ref)


def kernel(X, W, b):
    N, K = X.shape
    M = W.shape[1]
    bn = _BN if N % _BN == 0 else N
    y_hat, exit_points2d, exit_gate_logits = pl.pallas_call(
        _fused_kernel,
        grid=(N // bn,),
        in_specs=[
            pl.BlockSpec((bn, K), lambda i: (i, 0)),
            pl.BlockSpec((K, M), lambda i: (0, 0)),
            pl.BlockSpec((1, M), lambda i: (0, 0)),
        ],
        out_specs=[
            pl.BlockSpec((bn, M), lambda i: (i, 0)),
            pl.BlockSpec((bn, 1), lambda i: (i, 0)),
            pl.BlockSpec((bn, 1), lambda i: (i, 0)),
        ],
        out_shape=[
            jax.ShapeDtypeStruct((N, M), X.dtype),
            jax.ShapeDtypeStruct((N, 1), X.dtype),
            jax.ShapeDtypeStruct((N, 1), X.dtype),
        ],
    )(X, W, b.reshape(1, M))
    return (y_hat, exit_points2d.reshape(N), exit_gate_logits)


# speedup vs baseline: 1.1672x; 1.1672x over previous
import jax
import jax.numpy as jnp
from jax.experimental import pallas as pl

_BN = 4096


def _fused_kernel(x_ref, w_ref, b_ref, y_ref):
    acc = jnp.dot(x_ref[...], w_ref[...], preferred_element_type=jnp.float32)
    y_ref[...] = acc + b_ref[...]


def kernel(X, W, b):
    N, K = X.shape
    M = W.shape[1]
    bn = _BN if N % _BN == 0 else N
    y_hat = pl.pallas_call(
        _fused_kernel,
        grid=(N // bn,),
        in_specs=[
            pl.BlockSpec((bn, K), lambda i: (i, 0)),
            pl.BlockSpec((K, M), lambda i: (0, 0)),
            pl.BlockSpec((1, M), lambda i: (0, 0)),
        ],
        out_specs=pl.BlockSpec((bn, M), lambda i: (i, 0)),
        out_shape=jax.ShapeDtypeStruct((N, M), X.dtype),
    )(X, W, b.reshape(1, M))
    exit_points = jnp.zeros((N,), dtype=X.dtype)
    exit_gate_logits = jnp.zeros((N, 1), dtype=X.dtype)
    return (y_hat, exit_points, exit_gate_logits)
